# baseline (speedup 1.0000x reference)
import functools

import jax
import jax.numpy as jnp
from jax import lax
from jax.experimental import pallas as pl
from jax.experimental.pallas import tpu as pltpu
from jax.experimental.pallas import tpu_sc as plsc

NUM_FACTORS = 64
BATCH = 16384
NUM_ROWS = 1000000
BLK_FOLD = 8192
NQ = 8
QBLOCKS = 16
OFF8 = QBLOCKS * BLK_FOLD
NCOLB = (NUM_ROWS + BLK_FOLD - 1) // BLK_FOLD

NC, NS = 2, 16
NW = NC * NS
CHUNK = 128
B_PER_W = BATCH // NW
N_CHUNKS = B_PER_W // CHUNK


def _fold_body(t_ref0, t_ref1, t_ref2, t_ref3, t_ref4, t_ref5, t_ref6,
               t_ref7, w8_ref, out_ref):
    dn = (((0,), (0,)), ((), ()))
    x8 = jnp.concatenate(
        [t[...] for t in (t_ref0, t_ref1, t_ref2, t_ref3,
                          t_ref4, t_ref5, t_ref6, t_ref7)],
        axis=0).astype(jnp.bfloat16)
    h = lax.dot_general(x8, w8_ref[...], dn,
                        preferred_element_type=jnp.float32)
    a_bits = lax.bitcast_convert_type(h[:, :128], jnp.uint32)
    b_bits = lax.bitcast_convert_type(h[:, 128:], jnp.uint32)
    half = jnp.uint32(0x8000)
    lo = (a_bits + half) >> 16
    hi = (b_bits + half) & jnp.uint32(0xFFFF0000)
    out_ref[...] = lax.bitcast_convert_type(hi | lo, jnp.int32)


def _fold(tT, w8):
    def col_map(q):
        return lambda i: (0, jnp.minimum(q * QBLOCKS + i, NCOLB - 1))

    return pl.pallas_call(
        _fold_body,
        grid=(QBLOCKS,),
        in_specs=[pl.BlockSpec((NUM_FACTORS, BLK_FOLD), col_map(q))
                  for q in range(NQ)] +
                 [pl.BlockSpec((NQ * NUM_FACTORS, 256), lambda i: (0, 0))],
        out_specs=pl.BlockSpec((BLK_FOLD, 128), lambda i: (i, 0)),
        out_shape=jax.ShapeDtypeStruct((OFF8, 128), jnp.int32),
        compiler_params=pltpu.CompilerParams(
            fuse_transposed_lhs_in_matmul=True, skip_device_barrier=True),
    )(*([tT] * NQ), w8)


def _block_diag_w(w_half):
    z = jnp.zeros((NQ * NUM_FACTORS, 256), jnp.float32)
    for q in range(NQ):
        z = z.at[q * NUM_FACTORS:(q + 1) * NUM_FACTORS,
                 q * 32:(q + 1) * 32].set(w_half)
    return z.astype(jnp.bfloat16)


_sc_mesh = plsc.VectorSubcoreMesh(
    core_axis_name="c", subcore_axis_name="s", num_cores=NC, num_subcores=NS)


@functools.partial(
    pl.kernel,
    out_type=(
        jax.ShapeDtypeStruct((BATCH, 128), jnp.int32),
        jax.ShapeDtypeStruct((BATCH, 128), jnp.int32),
    ),
    mesh=_sc_mesh,
    scratch_types=[
        pltpu.VMEM((N_CHUNKS, CHUNK), jnp.int32),
        pltpu.VMEM((N_CHUNKS, CHUNK), jnp.int32),
        pltpu.VMEM((B_PER_W, 128), jnp.int32),
        pltpu.SemaphoreType.DMA,
    ],
)
def _sc_gather(uidx_hbm, iidx_hbm, hu_hbm, hi_hbm,
               u_out_hbm, i_out_hbm, uidx_v, iidx_v, rows_v, sem):
    wid = lax.axis_index("s") * NC + lax.axis_index("c")
    base = wid * B_PER_W
    pltpu.sync_copy(uidx_hbm.at[pl.ds(wid * N_CHUNKS, N_CHUNKS)], uidx_v)
    pltpu.sync_copy(iidx_hbm.at[pl.ds(wid * N_CHUNKS, N_CHUNKS)], iidx_v)
    for idx_v, h_hbm, out_hbm in ((uidx_v, hu_hbm, u_out_hbm),
                                  (iidx_v, hi_hbm, i_out_hbm)):
        copies = []
        for j in range(N_CHUNKS):
            copies.append(pltpu.async_copy(
                h_hbm.at[idx_v.at[j]],
                rows_v.at[pl.ds(j * CHUNK, CHUNK)], sem))
        for c in copies:
            c.wait()
        pltpu.sync_copy(rows_v, out_hbm.at[pl.ds(base, B_PER_W)])


_BLK_MLP = 4096


def _unpack_select(g_ref, m_ref):
    g = g_ref[...]
    a = lax.bitcast_convert_type(g << 16, jnp.float32)
    b = lax.bitcast_convert_type(
        lax.bitcast_convert_type(g, jnp.uint32) & jnp.uint32(0xFFFF0000),
        jnp.float32)
    m = m_ref[...]
    x = a * (m == 1).astype(jnp.float32) + b * (m == 2).astype(jnp.float32)
    return (x[:, 0:32] + x[:, 32:64]) + (x[:, 64:96] + x[:, 96:128])


def _mlp_body(gu_ref, gi_ref, mu_ref, mi_ref, b1_ref, w2_ref, b2_ref,
              w3_ref, b3_ref, out_ref):
    h = _unpack_select(gu_ref, mu_ref) + _unpack_select(gi_ref, mi_ref)
    h = jnp.maximum(h + b1_ref[...], 0.0)
    h2 = jnp.dot(h, w2_ref[...], preferred_element_type=jnp.float32)
    h2 = jnp.maximum(h2 + b2_ref[...], 0.0)
    out_ref[...] = jnp.dot(h2, w3_ref[...],
                           preferred_element_type=jnp.float32) + b3_ref[...]


def _mlp(gu, gi, mu, mi, b1r, W2, b2r, W3, b3r):
    full = lambda shape: pl.BlockSpec(shape, lambda i: (0, 0))
    return pl.pallas_call(
        _mlp_body,
        grid=(BATCH // _BLK_MLP,),
        in_specs=[
            pl.BlockSpec((_BLK_MLP, 128), lambda i: (i, 0)),
            pl.BlockSpec((_BLK_MLP, 128), lambda i: (i, 0)),
            pl.BlockSpec((_BLK_MLP, 128), lambda i: (i, 0)),
            pl.BlockSpec((_BLK_MLP, 128), lambda i: (i, 0)),
            full(b1r.shape), full(W2.shape), full(b2r.shape),
            full(W3.shape), full(b3r.shape),
        ],
        out_specs=pl.BlockSpec((_BLK_MLP, 1), lambda i: (i, 0)),
        out_shape=jax.ShapeDtypeStruct((BATCH, 1), jnp.float32),
        compiler_params=pltpu.CompilerParams(skip_device_barrier=True),
    )(gu, gi, mu, mi, b1r, W2, b2r, W3, b3r)


def _lane_mask(q8):
    lane_q = jnp.arange(128, dtype=jnp.int32)[None, :] // 32
    sel = lane_q == (q8 & 3)[:, None]
    stream = 1 + (q8 >> 2)[:, None]
    return jnp.where(sel, stream, 0).astype(jnp.int8)


def kernel(user, item, user_table, item_table, W1, b1, W2, b2, W3, b3):
    user = user.astype(jnp.int32)
    item = item.astype(jnp.int32)
    hu = _fold(user_table.T, _block_diag_w(W1[:NUM_FACTORS]))
    hi = _fold(item_table.T, _block_diag_w(W1[NUM_FACTORS:]))
    q8u = user >> 17
    q8i = item >> 17
    pu = (user & (OFF8 - 1)).reshape(BATCH // CHUNK, CHUNK)
    pi = (item & (OFF8 - 1)).reshape(BATCH // CHUNK, CHUNK)
    gu, gi = _sc_gather(pu, pi, hu, hi)
    return _mlp(gu, gi, _lane_mask(q8u), _lane_mask(q8i),
                b1.reshape(1, -1), W2, b2.reshape(1, -1), W3,
                b3.reshape(1, -1))
